# baseline (speedup 1.0000x reference)
import jax
import jax.numpy as jnp
from jax import lax
from jax.experimental import pallas as pl
from jax.experimental.pallas import tpu as pltpu
from jax.experimental.pallas import tpu_sc as plsc

B = 1024
C = 100000

_M = 0.4
_H = 0.333
_S = 64.0
_EPS = 0.001

_NC = 2
_NS = 16
_NW = _NC * _NS
_LPW = B // _NW

_CW = 11 * 128
_NCH = 71
_CSC = _CW * _NCH
_CTC = C - _CSC

_sc_mesh = plsc.VectorSubcoreMesh(
    core_axis_name="c", subcore_axis_name="s", num_cores=_NC, num_subcores=_NS
)

_LO = (-1.0 + _EPS) * _S
_HI = (1.0 - _EPS) * _S


def _consts_body(n_ref, cg_ref, sg_ref, clo_ref, chi_ref, ga_ref, gap_ref, cgp_ref):
    sn = jnp.clip(n_ref[...], 0.001, 100.0)
    mean_z = jnp.mean(sn)
    std_z = jnp.sqrt(jnp.sum((sn - mean_z) ** 2) / (B - 1))
    z = jnp.clip((sn - mean_z) / (std_z + _EPS) * _H, -1.0, 1.0)
    g = -_M * z
    cg = jnp.cos(g)
    sg = jnp.sin(g)
    clo = jnp.where(_EPS - g > 0, jnp.cos(_EPS - g), 2.0)
    chi = jnp.where(_EPS + g > 0, -jnp.cos(_EPS + g), -2.0)
    cgp = jnp.where(g >= 0, 2.0, cg)
    cg_ref[...] = cg
    sg_ref[...] = sg
    clo_ref[...] = clo
    chi_ref[...] = chi
    ga_ref[...] = _M + _M * z
    gap_ref[...] = 1.0 - _M * z - _M - jnp.cos(_M * z)
    cgp_ref[...] = cgp


def _consts_call(n8):
    shp = jax.ShapeDtypeStruct((8, 128), jnp.float32)
    return pl.pallas_call(
        _consts_body,
        out_shape=(shp,) * 7,
        name="adaact_consts",
    )(n8)


def _margin_from_consts(t, cg, sg, clo, chi, ga, gap, cgp, sqrt_fn):
    s_ = sqrt_fn(jnp.maximum(1.0 - t * t, 1e-12))
    cos_sum = t * cg - s_ * sg
    t_ang = jnp.where(t > clo, clo, jnp.where(t < chi, chi, cos_sum))
    t_add = t_ang - ga
    return jnp.where(t < cgp, t_add, t + gap)


def _newton_sqrt(x):
    y = 0.5 * x + 0.42
    for _ in range(7):
        y = 0.5 * (y + x / y)
    return y


def _sc_stream_body(x_hbm, lab_hbm, cg_hbm, sg_hbm, clo_hbm, chi_hbm, ga_hbm,
                    gap_hbm, cgp_hbm, o_hbm,
                    ina, inb, outa, outb, lab_v, cst_v,
                    isema, isemb, osema, osemb):
    wid = lax.axis_index("s") * _NC + lax.axis_index("c")
    base = pl.multiple_of(wid * _LPW, 32)
    ins = (ina, inb)
    outs = (outa, outb)
    isems = (isema, isemb)
    osems = (osema, osemb)

    pltpu.sync_copy(lab_hbm.at[pl.ds(base, _LPW)], lab_v.at[pl.ds(0, _LPW)])
    for j, ref in enumerate((cg_hbm, sg_hbm, clo_hbm, chi_hbm, ga_hbm,
                             gap_hbm, cgp_hbm)):
        pltpu.sync_copy(ref.at[pl.ds(base, _LPW)], cst_v.at[j, pl.ds(0, _LPW)])

    lane = lax.iota(jnp.int32, 16)

    def in_copy(r0, ch, k):
        return pltpu.make_async_copy(
            x_hbm.at[pl.ds(r0, 8), pl.ds(ch * _CW, _CW)], ins[k], isems[k]
        )

    def out_copy(r0, ch, k):
        return pltpu.make_async_copy(
            outs[k], o_hbm.at[pl.ds(r0, 8), pl.ds(ch * _CW, _CW)], osems[k]
        )

    @pl.loop(0, 4)
    def _stripe(st):
        r0 = pl.multiple_of(base + st * 8, 8)
        labs16 = lab_v[pl.ds(st * 8, 16)]
        cvecs = [cst_v[j, pl.ds(st * 8, 16)] for j in range(7)]
        in_copy(r0, 0, 0).start()
        in_copy(r0, 1, 1).start()

        @pl.loop(0, 36)
        def _pair(gh):
            g = gh * 2
            for b in range(2):
                ch = g + b

                @pl.when(ch < _NCH)
                def _():
                    in_copy(r0, ch, b).wait()

                    @pl.when(ch >= 2)
                    def _():
                        out_copy(r0, ch - 2, b).wait()

                    @pl.loop(0, _CW // 16, unroll=4)
                    def _vec(i):
                        sl = pl.ds(i * 16, 16)
                        for r in range(8):
                            outs[b][r, sl] = jnp.clip(
                                ins[b][r, sl] * _S, _LO, _HI
                            )

                    for r in range(8):
                        pos_r = labs16[r] - ch * _CW

                        @pl.when((pos_r >= 0) & (pos_r < _CW))
                        def _():
                            b16 = pl.multiple_of((pos_r >> 4) << 4, 16)
                            off = pos_r & 15
                            sl = pl.ds(b16, 16)
                            vec = outs[b][r, sl]
                            sel = lane == off
                            tv = jnp.where(sel, vec, 0.0) * (1.0 / _S)
                            cs = [cv[r] for cv in cvecs]
                            fv = _margin_from_consts(
                                tv, *cs, sqrt_fn=_newton_sqrt
                            )
                            outs[b][r, sl] = jnp.where(sel, fv * _S, vec)

                    out_copy(r0, ch, b).start()

                    @pl.when(ch + 2 < _NCH)
                    def _():
                        in_copy(r0, ch + 2, b).start()

        out_copy(r0, _NCH - 2, 1).wait()
        out_copy(r0, _NCH - 1, 0).wait()


_sc_stream = pl.kernel(
    _sc_stream_body,
    out_type=jax.ShapeDtypeStruct((B, C), jnp.float32),
    mesh=_sc_mesh,
    scratch_types=[
        pltpu.VMEM((8, _CW), jnp.float32),
        pltpu.VMEM((8, _CW), jnp.float32),
        pltpu.VMEM((8, _CW), jnp.float32),
        pltpu.VMEM((8, _CW), jnp.float32),
        pltpu.VMEM((64,), jnp.int32),
        pltpu.VMEM((7, 64), jnp.float32),
        pltpu.SemaphoreType.DMA,
        pltpu.SemaphoreType.DMA,
        pltpu.SemaphoreType.DMA,
        pltpu.SemaphoreType.DMA,
    ],
    compiler_params=pltpu.CompilerParams(use_tc_tiling_on_sc=True),
    name="adaact_sc_stream",
)


def _sliver_body(o_in_ref, x_ref, l_ref, cg_ref, sg_ref, clo_ref, chi_ref,
                 ga_ref, gap_ref, cgp_ref, o_ref):
    del o_in_ref
    y = jnp.clip(x_ref[...] * _S, _LO, _HI)
    iota = jax.lax.broadcasted_iota(jnp.int32, (B, 128), 1)
    mask = iota == (l_ref[...] - _CSC)
    t = jnp.sum(jnp.where(mask, y, 0.0), axis=1, keepdims=True) * (1.0 / _S)
    fv = _margin_from_consts(
        t, cg_ref[...], sg_ref[...], clo_ref[...], chi_ref[...], ga_ref[...],
        gap_ref[...], cgp_ref[...], sqrt_fn=jnp.sqrt,
    )
    o_ref[...] = jnp.where(mask, fv * _S, y)


def _sliver_call(sc_out, logits, labels_col, consts_col):
    colspec = pl.BlockSpec((B, 1), lambda i: (0, 0))
    return pl.pallas_call(
        _sliver_body,
        grid=(1,),
        in_specs=[
            pl.BlockSpec((B, 128), lambda i: (0, _CSC // 128)),
            pl.BlockSpec((B, 128), lambda i: (0, _CSC // 128)),
            colspec,
        ] + [colspec] * 7,
        out_specs=pl.BlockSpec((B, 128), lambda i: (0, _CSC // 128)),
        out_shape=jax.ShapeDtypeStruct((B, C), jnp.float32),
        input_output_aliases={0: 0},
        name="adaact_sliver",
    )(sc_out, logits, labels_col, *consts_col)


def kernel(logits, norms, labels):
    consts8 = _consts_call(norms.reshape(8, 128))
    consts_flat = [c.reshape(B) for c in consts8]
    sc_out = _sc_stream(logits, labels, *consts_flat)
    consts_col = [c.reshape(B, 1) for c in consts8]
    return _sliver_call(sc_out, logits, labels.reshape(B, 1), consts_col)


# speedup vs baseline: 1.3605x; 1.3605x over previous
import jax
import jax.numpy as jnp
from jax import lax
from jax.experimental import pallas as pl
from jax.experimental.pallas import tpu as pltpu
from jax.experimental.pallas import tpu_sc as plsc

B = 1024
C = 100000

_M = 0.4
_H = 0.333
_S = 64.0
_EPS = 0.001

_NC = 2
_NS = 16
_NW = _NC * _NS
_LPW = B // _NW

_CW = 11 * 128
_NCH = 71
_CSC = _CW * _NCH
_CTC = C - _CSC

_sc_mesh = plsc.VectorSubcoreMesh(
    core_axis_name="c", subcore_axis_name="s", num_cores=_NC, num_subcores=_NS
)

_LO = (-1.0 + _EPS) * _S
_HI = (1.0 - _EPS) * _S


def _consts_body(n_ref, cg_ref, sg_ref, clo_ref, chi_ref, ga_ref, gap_ref, cgp_ref):
    sn = jnp.clip(n_ref[...], 0.001, 100.0)
    mean_z = jnp.mean(sn)
    std_z = jnp.sqrt(jnp.sum((sn - mean_z) ** 2) / (B - 1))
    z = jnp.clip((sn - mean_z) / (std_z + _EPS) * _H, -1.0, 1.0)
    g = -_M * z
    cg = jnp.cos(g)
    sg = jnp.sin(g)
    clo = jnp.where(_EPS - g > 0, jnp.cos(_EPS - g), 2.0)
    chi = jnp.where(_EPS + g > 0, -jnp.cos(_EPS + g), -2.0)
    cgp = jnp.where(g >= 0, 2.0, cg)
    cg_ref[...] = cg
    sg_ref[...] = sg
    clo_ref[...] = clo
    chi_ref[...] = chi
    ga_ref[...] = _M + _M * z
    gap_ref[...] = 1.0 - _M * z - _M - jnp.cos(_M * z)
    cgp_ref[...] = cgp


def _consts_call(n8):
    shp = jax.ShapeDtypeStruct((8, 128), jnp.float32)
    return pl.pallas_call(
        _consts_body,
        out_shape=(shp,) * 7,
        name="adaact_consts",
    )(n8)


def _margin_from_consts(t, cg, sg, clo, chi, ga, gap, cgp, sqrt_fn):
    s_ = sqrt_fn(jnp.maximum(1.0 - t * t, 1e-12))
    cos_sum = t * cg - s_ * sg
    t_ang = jnp.where(t > clo, clo, jnp.where(t < chi, chi, cos_sum))
    t_add = t_ang - ga
    return jnp.where(t < cgp, t_add, t + gap)


def _newton_sqrt(x):
    y = 0.5 * x + 0.42
    for _ in range(7):
        y = 0.5 * (y + x / y)
    return y


def _sc_stream_body(x_hbm, lab_hbm, cg_hbm, sg_hbm, clo_hbm, chi_hbm, ga_hbm,
                    gap_hbm, cgp_hbm, o_hbm,
                    ina, inb, outa, outb, lab_v, cst_v,
                    isema, isemb, osema, osemb):
    wid = lax.axis_index("s") * _NC + lax.axis_index("c")
    base = pl.multiple_of(wid * _LPW, 32)
    ins = (ina, inb)
    outs = (outa, outb)
    isems = (isema, isemb)
    osems = (osema, osemb)

    pltpu.sync_copy(lab_hbm.at[pl.ds(base, _LPW)], lab_v.at[pl.ds(0, _LPW)])
    for j, ref in enumerate((cg_hbm, sg_hbm, clo_hbm, chi_hbm, ga_hbm,
                             gap_hbm, cgp_hbm)):
        pltpu.sync_copy(ref.at[pl.ds(base, _LPW)], cst_v.at[j, pl.ds(0, _LPW)])

    lane = lax.iota(jnp.int32, 16)

    def in_copy(r0, ch, k):
        return pltpu.make_async_copy(
            x_hbm.at[pl.ds(r0, 8), pl.ds(ch * _CW, _CW)], ins[k], isems[k]
        )

    def out_copy(r0, ch, k):
        return pltpu.make_async_copy(
            outs[k], o_hbm.at[pl.ds(r0, 8), pl.ds(ch * _CW, _CW)], osems[k]
        )

    @pl.loop(0, 4)
    def _stripe(st):
        r0 = pl.multiple_of(base + st * 8, 8)
        labs16 = lab_v[pl.ds(st * 8, 16)]
        cvecs = [cst_v[j, pl.ds(st * 8, 16)] for j in range(7)]
        in_copy(r0, 0, 0).start()
        in_copy(r0, 1, 1).start()

        @pl.loop(0, 36)
        def _pair(gh):
            g = gh * 2
            for b in range(2):
                ch = g + b

                @pl.when(ch < _NCH)
                def _():
                    in_copy(r0, ch, b).wait()

                    @pl.when(ch >= 2)
                    def _():
                        out_copy(r0, ch - 2, b).wait()

                    @pl.loop(0, _CW // 128)
                    def _tile(j):
                        tb = pl.multiple_of(j * 128, 128)
                        for r in range(8):
                            for q in range(8):
                                sl = pl.ds(tb + q * 16, 16)
                                outs[b][r, sl] = jnp.clip(
                                    ins[b][r, sl] * _S, _LO, _HI
                                )

                    for r in range(8):
                        pos_r = labs16[r] - ch * _CW

                        @pl.when((pos_r >= 0) & (pos_r < _CW))
                        def _():
                            b16 = pl.multiple_of((pos_r >> 4) << 4, 16)
                            off = pos_r & 15
                            sl = pl.ds(b16, 16)
                            vec = outs[b][r, sl]
                            sel = lane == off
                            tv = jnp.where(sel, vec, 0.0) * (1.0 / _S)
                            cs = [cv[r] for cv in cvecs]
                            fv = _margin_from_consts(
                                tv, *cs, sqrt_fn=_newton_sqrt
                            )
                            outs[b][r, sl] = jnp.where(sel, fv * _S, vec)

                    out_copy(r0, ch, b).start()

                    @pl.when(ch + 2 < _NCH)
                    def _():
                        in_copy(r0, ch + 2, b).start()

        out_copy(r0, _NCH - 2, 1).wait()
        out_copy(r0, _NCH - 1, 0).wait()


_sc_stream = pl.kernel(
    _sc_stream_body,
    out_type=jax.ShapeDtypeStruct((B, C), jnp.float32),
    mesh=_sc_mesh,
    scratch_types=[
        pltpu.VMEM((8, _CW), jnp.float32),
        pltpu.VMEM((8, _CW), jnp.float32),
        pltpu.VMEM((8, _CW), jnp.float32),
        pltpu.VMEM((8, _CW), jnp.float32),
        pltpu.VMEM((64,), jnp.int32),
        pltpu.VMEM((7, 64), jnp.float32),
        pltpu.SemaphoreType.DMA,
        pltpu.SemaphoreType.DMA,
        pltpu.SemaphoreType.DMA,
        pltpu.SemaphoreType.DMA,
    ],
    compiler_params=pltpu.CompilerParams(use_tc_tiling_on_sc=True),
    name="adaact_sc_stream",
)


def _sliver_body(o_in_ref, x_ref, l_ref, cg_ref, sg_ref, clo_ref, chi_ref,
                 ga_ref, gap_ref, cgp_ref, o_ref):
    del o_in_ref
    y = jnp.clip(x_ref[...] * _S, _LO, _HI)
    iota = jax.lax.broadcasted_iota(jnp.int32, (B, 128), 1)
    mask = iota == (l_ref[...] - _CSC)
    t = jnp.sum(jnp.where(mask, y, 0.0), axis=1, keepdims=True) * (1.0 / _S)
    fv = _margin_from_consts(
        t, cg_ref[...], sg_ref[...], clo_ref[...], chi_ref[...], ga_ref[...],
        gap_ref[...], cgp_ref[...], sqrt_fn=jnp.sqrt,
    )
    o_ref[...] = jnp.where(mask, fv * _S, y)


def _sliver_call(sc_out, logits, labels_col, consts_col):
    colspec = pl.BlockSpec((B, 1), lambda i: (0, 0))
    return pl.pallas_call(
        _sliver_body,
        grid=(1,),
        in_specs=[
            pl.BlockSpec((B, 128), lambda i: (0, _CSC // 128)),
            pl.BlockSpec((B, 128), lambda i: (0, _CSC // 128)),
            colspec,
        ] + [colspec] * 7,
        out_specs=pl.BlockSpec((B, 128), lambda i: (0, _CSC // 128)),
        out_shape=jax.ShapeDtypeStruct((B, C), jnp.float32),
        input_output_aliases={0: 0},
        name="adaact_sliver",
    )(sc_out, logits, labels_col, *consts_col)


def kernel(logits, norms, labels):
    consts8 = _consts_call(norms.reshape(8, 128))
    consts_flat = [c.reshape(B) for c in consts8]
    sc_out = _sc_stream(logits, labels, *consts_flat)
    consts_col = [c.reshape(B, 1) for c in consts8]
    return _sliver_call(sc_out, logits, labels.reshape(B, 1), consts_col)
